# baseline (speedup 1.0000x reference)
import functools

import jax
import jax.numpy as jnp
from jax import lax
from jax.experimental import pallas as pl
from jax.experimental.pallas import tpu as pltpu
from jax.experimental.pallas import tpu_sc as plsc

D = 1000
DP = 1008
B_TOTAL = 20480
NW = 32
B_PER_W = B_TOTAL // NW
CHUNK = 64
NCHUNK = B_PER_W // CHUNK


def _sc_gather(table, idx):
    mesh = plsc.VectorSubcoreMesh(core_axis_name="c", subcore_axis_name="s")

    @functools.partial(
        pl.kernel,
        mesh=mesh,
        compiler_params=pltpu.CompilerParams(use_tc_tiling_on_sc=False),
        out_type=jax.ShapeDtypeStruct((B_TOTAL, D), jnp.float32),
        scratch_types=[
            pltpu.VMEM((B_PER_W,), jnp.int32),
            pltpu.VMEM((2, CHUNK, DP), jnp.float32),
            pltpu.SemaphoreType.DMA,
            pltpu.SemaphoreType.DMA,
            pltpu.SemaphoreType.DMA,
            pltpu.SemaphoreType.DMA,
        ],
    )
    def k(table_hbm, idx_hbm, out_hbm, idx_v, rows_v, g0, g1, s0, s1):
        wid = lax.axis_index("s") * 2 + lax.axis_index("c")
        base = wid * B_PER_W
        gsem = (g0, g1)
        ssem = (s0, s1)
        pltpu.sync_copy(idx_hbm.at[pl.ds(base, B_PER_W)], idx_v)

        def gather(c, b):
            return pltpu.async_copy(
                table_hbm.at[idx_v.at[pl.ds(c * CHUNK, CHUNK)]],
                rows_v.at[b],
                gsem[b],
            )

        gathers = [gather(0, 0), None]
        scatters = [None, None]
        for c in range(NCHUNK):
            b = c % 2
            gathers[b].wait()
            if c + 1 < NCHUNK:
                nb = (c + 1) % 2
                if scatters[nb] is not None:
                    scatters[nb].wait()
                gathers[nb] = gather(c + 1, nb)
            scatters[b] = pltpu.async_copy(
                rows_v.at[b, :, pl.ds(0, D)],
                out_hbm.at[pl.ds(base + c * CHUNK, CHUNK)],
                ssem[b],
            )
        scatters[0].wait()
        scatters[1].wait()

    return k(table, idx)


def kernel(x, y, table):
    idx = x.reshape(-1).astype(jnp.int32)
    table_p = jnp.pad(table, ((0, 0), (0, DP - D)))
    out = _sc_gather(table_p, idx)
    return out.reshape(x.shape[0], x.shape[1], D)


# speedup vs baseline: 1.1214x; 1.1214x over previous
import functools

import jax
import jax.numpy as jnp
from jax import lax
from jax.experimental import pallas as pl
from jax.experimental.pallas import tpu as pltpu
from jax.experimental.pallas import tpu_sc as plsc

D = 1000
V = 1000
VP = 1024
B_TOTAL = 20480
NW = 32
NS = 16
B_PER_W = B_TOTAL // NW
CHUNK = 32
NCHUNK = B_PER_W // CHUNK
V_PER_S = VP // NS


def _sc_gather(table, idx):
    mesh = plsc.VectorSubcoreMesh(core_axis_name="c", subcore_axis_name="s")

    @functools.partial(
        pl.kernel,
        mesh=mesh,
        compiler_params=pltpu.CompilerParams(use_tc_tiling_on_sc=False),
        out_type=jax.ShapeDtypeStruct((B_TOTAL, D), jnp.float32),
        scratch_types=[
            pltpu.VMEM((B_PER_W,), jnp.int32),
            pltpu.VMEM((2, CHUNK, D), jnp.float32),
            pltpu.VMEM_SHARED((VP, D), jnp.float32),
            pltpu.SemaphoreType.DMA,
            pltpu.SemaphoreType.DMA,
            pltpu.SemaphoreType.DMA,
            pltpu.SemaphoreType.DMA,
        ],
    )
    def k(table_hbm, idx_hbm, out_hbm, idx_v, rows_v, table_s, g0, g1, s0, s1):
        sid = lax.axis_index("s")
        wid = sid * 2 + lax.axis_index("c")
        base = wid * B_PER_W
        gsem = (g0, g1)
        ssem = (s0, s1)
        vbase = sid * V_PER_S
        pltpu.sync_copy(
            table_hbm.at[pl.ds(vbase, V_PER_S)], table_s.at[pl.ds(vbase, V_PER_S)]
        )
        pltpu.sync_copy(idx_hbm.at[pl.ds(base, B_PER_W)], idx_v)
        plsc.subcore_barrier()

        def gather(c, b):
            return pltpu.async_copy(
                table_s.at[idx_v.at[pl.ds(c * CHUNK, CHUNK)]],
                rows_v.at[b],
                gsem[b],
            )

        gathers = [gather(0, 0), None]
        scatters = [None, None]
        for c in range(NCHUNK):
            b = c % 2
            gathers[b].wait()
            if c + 1 < NCHUNK:
                nb = (c + 1) % 2
                if scatters[nb] is not None:
                    scatters[nb].wait()
                gathers[nb] = gather(c + 1, nb)
            scatters[b] = pltpu.async_copy(
                rows_v.at[b],
                out_hbm.at[pl.ds(base + c * CHUNK, CHUNK)],
                ssem[b],
            )
        scatters[0].wait()
        scatters[1].wait()

    return k(table, idx)


def kernel(x, y, table):
    idx = x.reshape(-1).astype(jnp.int32)
    table_p = jnp.pad(table, ((0, VP - V), (0, 0)))
    out = _sc_gather(table_p, idx)
    return out.reshape(x.shape[0], x.shape[1], D)


# speedup vs baseline: 1.1219x; 1.0005x over previous
import functools

import jax
import jax.numpy as jnp
from jax import lax
from jax.experimental import pallas as pl
from jax.experimental.pallas import tpu as pltpu
from jax.experimental.pallas import tpu_sc as plsc

D = 1000
V = 1000
VP = 1024
B_TOTAL = 20480
NW = 32
NS = 16
B_PER_W = B_TOTAL // NW
CHUNK = 32
NCHUNK = B_PER_W // CHUNK
V_PER_S = VP // NS


def _sc_gather(table, idx):
    mesh = plsc.VectorSubcoreMesh(core_axis_name="c", subcore_axis_name="s")

    @functools.partial(
        pl.kernel,
        mesh=mesh,
        compiler_params=pltpu.CompilerParams(use_tc_tiling_on_sc=False),
        out_type=jax.ShapeDtypeStruct((B_TOTAL, D), jnp.float32),
        scratch_types=[
            pltpu.VMEM((B_PER_W,), jnp.int32),
            pltpu.VMEM((2, CHUNK, D), jnp.float32),
            pltpu.VMEM_SHARED((V, D), jnp.float32),
            pltpu.SemaphoreType.DMA,
            pltpu.SemaphoreType.DMA,
            pltpu.SemaphoreType.DMA,
            pltpu.SemaphoreType.DMA,
        ],
    )
    def k(table_hbm, idx_hbm, out_hbm, idx_v, rows_v, table_s, g0, g1, s0, s1):
        sid = lax.axis_index("s")
        wid = sid * 2 + lax.axis_index("c")
        base = wid * B_PER_W
        gsem = (g0, g1)
        ssem = (s0, s1)
        @pl.when(sid < 8)
        def _stage():
            vbase = sid * 125
            pltpu.sync_copy(
                table_hbm.at[pl.ds(vbase, 125)], table_s.at[pl.ds(vbase, 125)]
            )

        pltpu.sync_copy(idx_hbm.at[pl.ds(base, B_PER_W)], idx_v)
        plsc.subcore_barrier()

        def gather(c, b):
            return pltpu.async_copy(
                table_s.at[idx_v.at[pl.ds(c * CHUNK, CHUNK)]],
                rows_v.at[b],
                gsem[b],
            )

        gathers = [gather(0, 0), None]
        scatters = [None, None]
        for c in range(NCHUNK):
            b = c % 2
            gathers[b].wait()
            if c + 1 < NCHUNK:
                nb = (c + 1) % 2
                if scatters[nb] is not None:
                    scatters[nb].wait()
                gathers[nb] = gather(c + 1, nb)
            scatters[b] = pltpu.async_copy(
                rows_v.at[b],
                out_hbm.at[pl.ds(base + c * CHUNK, CHUNK)],
                ssem[b],
            )
        scatters[0].wait()
        scatters[1].wait()

    return k(table, idx)


def kernel(x, y, table):
    idx = x.reshape(-1).astype(jnp.int32)
    out = _sc_gather(table, idx)
    return out.reshape(x.shape[0], x.shape[1], D)
